# Initial kernel scaffold; baseline (speedup 1.0000x reference)
import jax
import jax.numpy as jnp
from jax.experimental import pallas as pl


def kernel(h, edge_index, e, W_enc, b_enc, W_self, W_neigh, b_self, b_neigh, bn_gamma, bn_beta, bn_mean, bn_var, bnh_gamma, bnh_beta, bnh_mean, bnh_var, W_r1, b_r1, W_r2, b_r2, W_r3, b_r3):
    raise NotImplementedError("write your pallas kernel here")


# speedup vs baseline: 3.7177x; 3.7177x over previous
import functools

import jax
import jax.numpy as jnp
from jax import lax
from jax.experimental import pallas as pl
from jax.experimental.pallas import tpu as pltpu
from jax.experimental.pallas import tpu_sc as plsc

NN = 10000
NP = 10240
DD = 128
EE = 320000
NCORE = 2
NSUB = 16
NW = NCORE * NSUB
EW = 10112
EPAD = NW * EW
CHUNK = 128
NCHUNK = EW // CHUNK
ROWS_PER_TILE = NP // NSUB


def _sc_degree_body(dst_hbm, ones_hbm, z128_hbm, degp_hbm, acc, ones_v, idx_v):
    c = lax.axis_index("c")
    s = lax.axis_index("s")
    wid = s * NCORE + c
    row0 = s * ROWS_PER_TILE
    pltpu.sync_copy(z128_hbm, acc.at[pl.ds(row0, ROWS_PER_TILE)])
    pltpu.sync_copy(ones_hbm, ones_v)
    plsc.subcore_barrier()
    base = wid * EW

    def body(i, carry):
        off = pl.multiple_of(base + i * CHUNK, CHUNK)
        pltpu.sync_copy(dst_hbm.at[pl.ds(off, CHUNK)], idx_v)
        pltpu.sync_copy(ones_v, acc.at[idx_v], add=True)
        return carry

    lax.fori_loop(0, NCHUNK, body, 0)
    plsc.subcore_barrier()
    pltpu.sync_copy(acc.at[pl.ds(row0, ROWS_PER_TILE)],
                    degp_hbm.at[c, pl.ds(row0, ROWS_PER_TILE)])


def _sc_segsum_body(hn_hbm, src_hbm, dst_hbm, z128_hbm, accp_hbm,
                    acc, rows_v, src_v, dst_v, sem):
    c = lax.axis_index("c")
    s = lax.axis_index("s")
    wid = s * NCORE + c
    row0 = s * ROWS_PER_TILE
    pltpu.sync_copy(z128_hbm, acc.at[pl.ds(row0, ROWS_PER_TILE)])
    plsc.subcore_barrier()
    base = wid * EW

    def body(i, carry):
        off = pl.multiple_of(base + i * CHUNK, CHUNK)
        pltpu.sync_copy(src_hbm.at[pl.ds(off, CHUNK)], src_v)
        pltpu.sync_copy(dst_hbm.at[pl.ds(off, CHUNK)], dst_v)
        pltpu.async_copy(hn_hbm.at[src_v], rows_v, sem).wait()
        pltpu.sync_copy(rows_v, acc.at[dst_v], add=True)
        return carry

    lax.fori_loop(0, NCHUNK, body, 0)
    plsc.subcore_barrier()
    pltpu.sync_copy(acc.at[pl.ds(row0, ROWS_PER_TILE)],
                    accp_hbm.at[c, pl.ds(row0, ROWS_PER_TILE)])


@functools.cache
def _sc_kernels():
    mesh = plsc.VectorSubcoreMesh(core_axis_name="c", subcore_axis_name="s")
    deg = pl.kernel(
        _sc_degree_body,
        out_type=jax.ShapeDtypeStruct((NCORE, NP, DD), jnp.float32),
        mesh=mesh,
        scratch_types=[
            pltpu.VMEM_SHARED((NP, DD), jnp.float32),
            pltpu.VMEM((CHUNK, DD), jnp.float32),
            pltpu.VMEM((CHUNK,), jnp.int32),
        ],
    )
    seg = pl.kernel(
        _sc_segsum_body,
        out_type=jax.ShapeDtypeStruct((NCORE, NP, DD), jnp.float32),
        mesh=mesh,
        scratch_types=[
            pltpu.VMEM_SHARED((NP, DD), jnp.float32),
            pltpu.VMEM((CHUNK, DD), jnp.float32),
            pltpu.VMEM((CHUNK,), jnp.int32),
            pltpu.VMEM((CHUNK,), jnp.int32),
            pltpu.SemaphoreType.DMA,
        ],
    )
    return deg, seg


def _bn_affine(g, b, m, v):
    scale = g * lax.rsqrt(v + 1e-5)
    return scale, b - m * scale


def _row_mask(x):
    keep = lax.broadcasted_iota(jnp.int32, (NP, 1), 0) < NN
    return jnp.where(keep, x, 0.0)


def _enc_body(h_ref, we_ref, be_ref, degp_ref, g_ref, b_ref, m_ref, v_ref,
              h0_ref, h0n_ref, norm_ref, cs_ref):
    h0 = jnp.dot(h_ref[...], we_ref[...],
                 preferred_element_type=jnp.float32) + be_ref[...]
    deg = jnp.maximum(degp_ref[0][:, 0:16] + degp_ref[1][:, 0:16], 1.0)
    norm16 = lax.rsqrt(deg)
    norm_ref[...] = norm16
    h0_ref[...] = h0
    h0n_ref[...] = h0 * norm16[:, 0:1]
    scale, shift = _bn_affine(g_ref[...], b_ref[...], m_ref[...], v_ref[...])
    z = jnp.maximum(jnp.maximum(h0, 0.0) * scale + shift, 0.0)
    cs_ref[...] = jnp.sum(_row_mask(z), axis=0, keepdims=True)


def _make_layer_body(emit_hn):
    def body(h_ref, accp_ref, norm_ref, ws_ref, wn_ref, bias_ref,
             g_ref, b_ref, m_ref, v_ref, hg_ref, hb_ref, hm_ref, hv_ref,
             *out_refs):
        nrm = norm_ref[...][:, 0:1]
        agg = (accp_ref[0] + accp_ref[1]) * nrm
        x = (jnp.dot(h_ref[...], ws_ref[...], preferred_element_type=jnp.float32)
             + jnp.dot(agg, wn_ref[...], preferred_element_type=jnp.float32)
             + bias_ref[...])
        scale, shift = _bn_affine(g_ref[...], b_ref[...], m_ref[...], v_ref[...])
        hnew = jnp.maximum(x * scale + shift, 0.0)
        out_refs[0][...] = hnew
        hs, hh = _bn_affine(hg_ref[...], hb_ref[...], hm_ref[...], hv_ref[...])
        z = jnp.maximum(hnew * hs + hh, 0.0)
        out_refs[1][...] = jnp.sum(_row_mask(z), axis=0, keepdims=True)
        if emit_hn:
            out_refs[2][...] = hnew * nrm
    return body


def _readout_body(c0_ref, c1_ref, c2_ref, c3_ref, w1_ref, b1_ref,
                  w2_ref, b2_ref, w3_ref, b3_ref, out_ref):
    hg = jnp.concatenate(
        [c0_ref[...], c1_ref[...], c2_ref[...], c3_ref[...]], axis=1) / NN
    o = jnp.maximum(jnp.dot(hg, w1_ref[...],
                            preferred_element_type=jnp.float32) + b1_ref[...], 0.0)
    o = jnp.maximum(jnp.dot(o, w2_ref[...],
                            preferred_element_type=jnp.float32) + b2_ref[...], 0.0)
    out_ref[...] = jnp.dot(o, w3_ref[...],
                           preferred_element_type=jnp.float32) + b3_ref[...]


def _f32(*shape):
    return jax.ShapeDtypeStruct(shape, jnp.float32)


_enc_call = pl.pallas_call(
    _enc_body,
    out_shape=[_f32(NP, DD), _f32(NP, DD), _f32(NP, 16), _f32(1, DD)],
)

_layer_call_mid = pl.pallas_call(
    _make_layer_body(True),
    out_shape=[_f32(NP, DD), _f32(1, DD), _f32(NP, DD)],
)

_layer_call_last = pl.pallas_call(
    _make_layer_body(False),
    out_shape=[_f32(NP, DD), _f32(1, DD)],
)

_readout_call = pl.pallas_call(_readout_body, out_shape=_f32(1, 10))


def kernel(h, edge_index, e, W_enc, b_enc, W_self, W_neigh, b_self, b_neigh,
           bn_gamma, bn_beta, bn_mean, bn_var, bnh_gamma, bnh_beta, bnh_mean,
           bnh_var, W_r1, b_r1, W_r2, b_r2, W_r3, b_r3):
    del e
    src = edge_index[0]
    dst = edge_index[1]
    npad = EPAD - EE
    src_p = jnp.concatenate([src, jnp.zeros((npad,), jnp.int32)])
    dst_p = jnp.concatenate([dst, jnp.full((npad,), NP - 1, jnp.int32)])
    h_pad = jnp.zeros((NP, DD), jnp.float32).at[:NN].set(h)
    ones128 = jnp.ones((CHUNK, DD), jnp.float32)
    z128 = jnp.zeros((ROWS_PER_TILE, DD), jnp.float32)

    def seg(sidx):
        sl = slice(sidx * DD, (sidx + 1) * DD)
        return (bnh_gamma[sl].reshape(1, DD), bnh_beta[sl].reshape(1, DD),
                bnh_mean[sl].reshape(1, DD), bnh_var[sl].reshape(1, DD))

    _sc_degree, _sc_segsum = _sc_kernels()
    degp = _sc_degree(dst_p, ones128, z128)
    h_cur, hn, norm16, cs0 = _enc_call(
        h_pad, W_enc, b_enc.reshape(1, DD), degp, *seg(0))

    colsums = [cs0]
    for l in range(3):
        accp = _sc_segsum(hn, src_p, dst_p, z128)
        bias = (b_self[l] + b_neigh[l]).reshape(1, DD)
        args = (h_cur, accp, norm16, W_self[l], W_neigh[l], bias,
                bn_gamma[l].reshape(1, DD), bn_beta[l].reshape(1, DD),
                bn_mean[l].reshape(1, DD), bn_var[l].reshape(1, DD),
                *seg(l + 1))
        if l < 2:
            h_cur, cs, hn = _layer_call_mid(*args)
        else:
            h_cur, cs = _layer_call_last(*args)
        colsums.append(cs)

    return _readout_call(*colsums, W_r1, b_r1.reshape(1, -1),
                         W_r2, b_r2.reshape(1, -1), W_r3, b_r3.reshape(1, -1))
